# baseline (speedup 1.0000x reference)
import jax
import jax.numpy as jnp
from jax import lax
from jax.experimental import pallas as pl
from jax.experimental.pallas import tpu as pltpu
from jax.experimental.pallas import tpu_sc as plsc

B, H, W, L = 8, 90, 256, 256
WL = W * L
NC, NS, LANES = 2, 16, 16
NW = NC * NS
NQ = NW // B
CH = WL // NQ
NV = CH // LANES


def _sc_body(cs_hbm, csp_hbm, m_hbm, out_hbm,
             mbuf, a0, a1, p0, p1, obuf, sa0, sp0, sa1, sp1):
    c = lax.axis_index("c")
    s = lax.axis_index("s")
    wid = s * NC + c
    b = wid // NQ
    qoff = (wid % NQ) * CH

    pltpu.sync_copy(m_hbm.at[b, pl.ds(qoff, CH)], mbuf)

    def _binm(i, carry):
        m = mbuf[pl.ds(i * LANES, LANES)]
        mbuf[pl.ds(i * LANES, LANES)] = jnp.where(m > 0.0, 1.0, 0.0)
        return carry
    lax.fori_loop(0, NV, _binm, 0)

    pltpu.async_copy(cs_hbm.at[b, 0, pl.ds(qoff, CH)], a0, sa0)
    pltpu.async_copy(csp_hbm.at[b, 0, pl.ds(qoff, CH)], p0, sp0)
    pltpu.async_copy(cs_hbm.at[b, 1, pl.ds(qoff, CH)], a1, sa1)
    pltpu.async_copy(csp_hbm.at[b, 1, pl.ds(qoff, CH)], p1, sp1)

    def _chunk(abuf, pbuf, acc):
        def _inner(i, acc):
            a = abuf[pl.ds(i * LANES, LANES)]
            p = pbuf[pl.ds(i * LANES, LANES)]
            m = mbuf[pl.ds(i * LANES, LANES)]
            a = jnp.where(jnp.isnan(a), 0.0, a)
            d = a - p * m
            return acc + d * d
        return lax.fori_loop(0, NV, _inner, acc, unroll=8)

    def _outer(k, acc):
        h0 = 2 * k
        pltpu.make_async_copy(cs_hbm.at[b, h0, pl.ds(qoff, CH)], a0, sa0).wait()
        pltpu.make_async_copy(csp_hbm.at[b, h0, pl.ds(qoff, CH)], p0, sp0).wait()
        acc = _chunk(a0, p0, acc)

        @pl.when(h0 + 2 < H)
        def _():
            pltpu.async_copy(cs_hbm.at[b, h0 + 2, pl.ds(qoff, CH)], a0, sa0)
            pltpu.async_copy(csp_hbm.at[b, h0 + 2, pl.ds(qoff, CH)], p0, sp0)

        pltpu.make_async_copy(cs_hbm.at[b, h0 + 1, pl.ds(qoff, CH)], a1, sa1).wait()
        pltpu.make_async_copy(csp_hbm.at[b, h0 + 1, pl.ds(qoff, CH)], p1, sp1).wait()
        acc = _chunk(a1, p1, acc)

        @pl.when(h0 + 3 < H)
        def _():
            pltpu.async_copy(cs_hbm.at[b, h0 + 3, pl.ds(qoff, CH)], a1, sa1)
            pltpu.async_copy(csp_hbm.at[b, h0 + 3, pl.ds(qoff, CH)], p1, sp1)
        return acc

    acc = lax.fori_loop(0, H // 2, _outer, jnp.zeros((LANES,), jnp.float32))
    obuf[...] = acc
    pltpu.sync_copy(obuf, out_hbm.at[wid])


_mesh = plsc.VectorSubcoreMesh(core_axis_name="c", subcore_axis_name="s")

_sc_call = pl.kernel(
    _sc_body,
    out_type=jax.ShapeDtypeStruct((NW, LANES), jnp.float32),
    mesh=_mesh,
    scratch_types=[
        pltpu.VMEM((CH,), jnp.float32),
        pltpu.VMEM((CH,), jnp.float32),
        pltpu.VMEM((CH,), jnp.float32),
        pltpu.VMEM((CH,), jnp.float32),
        pltpu.VMEM((CH,), jnp.float32),
        pltpu.VMEM((LANES,), jnp.float32),
        pltpu.SemaphoreType.DMA,
        pltpu.SemaphoreType.DMA,
        pltpu.SemaphoreType.DMA,
        pltpu.SemaphoreType.DMA,
    ],
)


def kernel(cs, cs_p, overpass_mask):
    csp3 = cs_p.reshape(B, H, WL)
    m2 = overpass_mask.reshape(B, WL)
    partials = _sc_call(cs, csp3, m2)
    return jnp.sum(partials) / jnp.float32(B * H * WL)
